# baseline (speedup 1.0000x reference)
import functools

import jax
import jax.numpy as jnp
from jax import lax
from jax.experimental import pallas as pl
from jax.experimental.pallas import tpu as pltpu
from jax.experimental.pallas import tpu_sc as plsc

N_NODES = 10000
N_EDGES = 320000
HID = 128
LANES = 16
NC = 2
NS = 16
NW = NC * NS


_sc_mesh = plsc.VectorSubcoreMesh(core_axis_name="c", subcore_axis_name="s",
                                  num_cores=NC, num_subcores=NS)


def _make_gather(n_edges, ch):
    epw = n_edges // NW
    nch = epw // ch
    assert epw % ch == 0 and ch % 8 == 0 and ch <= 128

    def body(ps_hbm, pr_hbm, s2d_hbm, r2d_hbm, out_hbm,
             idx_s, idx_r, rows_s0, rows_r0, rows_s1, rows_r1,
             obuf0, obuf1,
             sem_s0, sem_r0, sem_s1, sem_r1, sem_w0, sem_w1):
        wid = lax.axis_index("s") * NC + lax.axis_index("c")
        base = wid * epw

        rows_s = (rows_s0, rows_s1)
        rows_r = (rows_r0, rows_r1)
        obuf = (obuf0, obuf1)
        sem_s = (sem_s0, sem_s1)
        sem_r = (sem_r0, sem_r1)
        sem_w = (sem_w0, sem_w1)

        pltpu.sync_copy(s2d_hbm.at[wid], idx_s)
        pltpu.sync_copy(r2d_hbm.at[wid], idx_r)

        def g_start(j, b):
            pltpu.make_async_copy(ps_hbm.at[idx_s.at[j]], rows_s[b], sem_s[b]).start()
            pltpu.make_async_copy(pr_hbm.at[idx_r.at[j]], rows_r[b], sem_r[b]).start()

        def g_wait(j, b):
            pltpu.make_async_copy(ps_hbm.at[idx_s.at[j]], rows_s[b], sem_s[b]).wait()
            pltpu.make_async_copy(pr_hbm.at[idx_r.at[j]], rows_r[b], sem_r[b]).wait()

        def w_start(j, b):
            pltpu.make_async_copy(obuf[b], out_hbm.at[pl.ds(base + j * ch, ch)],
                                  sem_w[b]).start()

        def w_wait(j, b):
            pltpu.make_async_copy(obuf[b], out_hbm.at[pl.ds(base + j * ch, ch)],
                                  sem_w[b]).wait()

        g_start(0, 0)

        def pair(jj, carry):
            for b in range(2):
                j = 2 * jj + b

                @pl.when(j + 1 < nch)
                def _():
                    g_start(j + 1, 1 - b)

                @pl.when(j < nch)
                def _():
                    g_wait(j, b)

                    @pl.when(j >= 2)
                    def _():
                        w_wait(j - 2, b)

                    def addrow(i, c2):
                        for k in range(HID // LANES):
                            sl = pl.ds(k * LANES, LANES)
                            obuf[b][i, sl] = rows_s[b][i, sl] + rows_r[b][i, sl]
                        return c2

                    lax.fori_loop(0, ch, addrow, 0)
                    w_start(j, b)
            return carry

        lax.fori_loop(0, (nch + 1) // 2, pair, 0)
        w_wait(nch - 2, nch % 2)
        w_wait(nch - 1, 1 - nch % 2)

    return functools.partial(
        pl.kernel,
        out_type=jax.ShapeDtypeStruct((n_edges, HID), jnp.float32),
        mesh=_sc_mesh,
        scratch_types=[
            pltpu.VMEM((nch, ch), jnp.int32),
            pltpu.VMEM((nch, ch), jnp.int32),
            pltpu.VMEM((ch, HID), jnp.float32),
            pltpu.VMEM((ch, HID), jnp.float32),
            pltpu.VMEM((ch, HID), jnp.float32),
            pltpu.VMEM((ch, HID), jnp.float32),
            pltpu.VMEM((ch, HID), jnp.float32),
            pltpu.VMEM((ch, HID), jnp.float32),
            pltpu.SemaphoreType.DMA,
            pltpu.SemaphoreType.DMA,
            pltpu.SemaphoreType.DMA,
            pltpu.SemaphoreType.DMA,
            pltpu.SemaphoreType.DMA,
            pltpu.SemaphoreType.DMA,
        ],
    )(body)


_NPAD = 10240
_RPT = _NPAD // NS


def _make_segsum(n_edges, ch):
    ept = n_edges // NS
    snch = ept // ch
    assert ept % ch == 0 and snch % 2 == 0 and ch % 8 == 0 and ch <= 128

    def body(enew_hbm, idxcat_hbm, out_hbm, idx0, idx1, rows0, rows1, acc,
             sem_i0, sem_i1, sem_l0, sem_l1, sem_a0, sem_a1):
        c = lax.axis_index("c")
        s = lax.axis_index("s")

        idx = (idx0, idx1)
        rows = (rows0, rows1)
        sem_i = (sem_i0, sem_i1)
        sem_l = (sem_l0, sem_l1)
        sem_a = (sem_a0, sem_a1)

        zv = jnp.zeros((LANES,), jnp.float32)

        def zrow(i, c2):
            for k in range(HID // LANES):
                rows0[i, pl.ds(k * LANES, LANES)] = zv
            return c2

        lax.fori_loop(0, ch, zrow, 0)
        for k in range(_RPT // ch):
            pltpu.sync_copy(rows0, acc.at[pl.ds(s * _RPT + k * ch, ch)])
        plsc.subcore_barrier()

        def l_start(j, b):
            pltpu.make_async_copy(
                idxcat_hbm.at[pl.ds(c * n_edges + s * ept + j * ch, ch)],
                idx[b], sem_i[b]).start()
            pltpu.make_async_copy(enew_hbm.at[pl.ds(s * ept + j * ch, ch)],
                                  rows[b], sem_l[b]).start()

        def l_wait(j, b):
            pltpu.make_async_copy(
                idxcat_hbm.at[pl.ds(c * n_edges + s * ept + j * ch, ch)],
                idx[b], sem_i[b]).wait()
            pltpu.make_async_copy(enew_hbm.at[pl.ds(s * ept + j * ch, ch)],
                                  rows[b], sem_l[b]).wait()

        def a_start(j, b):
            pltpu.async_copy(rows[b], acc.at[idx[b]], sem_a[b], add=True)

        def a_wait(j, b):
            pltpu.make_async_copy(rows[b], acc.at[idx[b]], sem_a[b]).wait()

        l_start(0, 0)

        def pair(jj, carry):
            for b in range(2):
                j = 2 * jj + b

                @pl.when(j + 1 < snch)
                def _():
                    @pl.when(j >= 1)
                    def _():
                        a_wait(j - 1, 1 - b)

                    l_start(j + 1, 1 - b)

                l_wait(j, b)
                a_start(j, b)
            return carry

        lax.fori_loop(0, snch // 2, pair, 0)
        a_wait(snch - 2, 0)
        a_wait(snch - 1, 1)
        plsc.subcore_barrier()

        @pl.when(s < NS - 1)
        def _():
            pltpu.sync_copy(acc.at[pl.ds(s * _RPT, _RPT)],
                            out_hbm.at[c, pl.ds(s * _RPT, _RPT)])

        @pl.when(s == NS - 1)
        def _():
            last = N_NODES - (NS - 1) * _RPT
            pltpu.sync_copy(acc.at[pl.ds((NS - 1) * _RPT, last)],
                            out_hbm.at[c, pl.ds((NS - 1) * _RPT, last)])

    return functools.partial(
        pl.kernel,
        out_type=jax.ShapeDtypeStruct((2, N_NODES, HID), jnp.float32),
        mesh=_sc_mesh,
        scratch_types=[
            pltpu.VMEM((ch,), jnp.int32),
            pltpu.VMEM((ch,), jnp.int32),
            pltpu.VMEM((ch, HID), jnp.float32),
            pltpu.VMEM((ch, HID), jnp.float32),
            pltpu.VMEM_SHARED((_NPAD, HID), jnp.float32),
            pltpu.SemaphoreType.DMA,
            pltpu.SemaphoreType.DMA,
            pltpu.SemaphoreType.DMA,
            pltpu.SemaphoreType.DMA,
            pltpu.SemaphoreType.DMA,
            pltpu.SemaphoreType.DMA,
        ],
    )(body)


def _node_embed_body(nf_ref, w_ref, b_ref, ws_ref, wr_ref, x_ref, ps_ref, pr_ref):
    x = jnp.dot(nf_ref[...], w_ref[...], preferred_element_type=jnp.float32)
    x = x + b_ref[...]
    x_ref[...] = x
    ps_ref[...] = jnp.dot(x, ws_ref[...], preferred_element_type=jnp.float32)
    pr_ref[...] = jnp.dot(x, wr_ref[...], preferred_element_type=jnp.float32)


def _node_embed(nf, w, b, ws, wr):
    return pl.pallas_call(
        _node_embed_body,
        out_shape=[
            jax.ShapeDtypeStruct((N_NODES, HID), jnp.float32),
            jax.ShapeDtypeStruct((N_NODES, HID), jnp.float32),
            jax.ShapeDtypeStruct((N_NODES, HID), jnp.float32),
        ],
    )(nf, w, b.reshape(1, HID), ws, wr)


_EB = 4000


def _ln(r, g, b):
    m = jnp.mean(r, axis=-1, keepdims=True)
    d = r - m
    v = jnp.mean(d * d, axis=-1, keepdims=True)
    return d * lax.rsqrt(v + 1e-6) * g + b


def _edge_update_body(e_ref, g_ref, w_ref, b_ref, lg_ref, lb_ref,
                      eo_ref, en_ref):
    e = e_ref[...]
    en = (jnp.dot(e, w_ref[...], preferred_element_type=jnp.float32)
          + g_ref[...] + b_ref[...])
    en_ref[...] = en
    eo_ref[...] = _ln(e + en, lg_ref[...], lb_ref[...])


def _edge_update(e, g, w, b, lg, lb):
    return pl.pallas_call(
        _edge_update_body,
        grid=(e.shape[0] // _EB,),
        in_specs=[
            pl.BlockSpec((_EB, HID), lambda i: (i, 0)),
            pl.BlockSpec((_EB, HID), lambda i: (i, 0)),
            pl.BlockSpec((HID, HID), lambda i: (0, 0)),
            pl.BlockSpec((1, HID), lambda i: (0, 0)),
            pl.BlockSpec((1, HID), lambda i: (0, 0)),
            pl.BlockSpec((1, HID), lambda i: (0, 0)),
        ],
        out_specs=[
            pl.BlockSpec((_EB, HID), lambda i: (i, 0)),
            pl.BlockSpec((_EB, HID), lambda i: (i, 0)),
        ],
        out_shape=[
            jax.ShapeDtypeStruct((e.shape[0], HID), jnp.float32),
            jax.ShapeDtypeStruct((e.shape[0], HID), jnp.float32),
        ],
    )(e, g, w, b.reshape(1, HID), lg.reshape(1, HID), lb.reshape(1, HID))


def _edge_update_first_body(ea_ref, we_ref, be_ref, g_ref, w_ref, b_ref,
                            lg_ref, lb_ref, eo_ref, en_ref):
    e = (jnp.dot(ea_ref[...], we_ref[...], preferred_element_type=jnp.float32)
         + be_ref[...])
    en = (jnp.dot(e, w_ref[...], preferred_element_type=jnp.float32)
          + g_ref[...] + b_ref[...])
    en_ref[...] = en
    eo_ref[...] = _ln(e + en, lg_ref[...], lb_ref[...])


def _edge_update_first(ea, we, be, g, w, b, lg, lb):
    return pl.pallas_call(
        _edge_update_first_body,
        grid=(ea.shape[0] // _EB,),
        in_specs=[
            pl.BlockSpec((_EB, 16), lambda i: (i, 0)),
            pl.BlockSpec((16, HID), lambda i: (0, 0)),
            pl.BlockSpec((1, HID), lambda i: (0, 0)),
            pl.BlockSpec((_EB, HID), lambda i: (i, 0)),
            pl.BlockSpec((HID, HID), lambda i: (0, 0)),
            pl.BlockSpec((1, HID), lambda i: (0, 0)),
            pl.BlockSpec((1, HID), lambda i: (0, 0)),
            pl.BlockSpec((1, HID), lambda i: (0, 0)),
        ],
        out_specs=[
            pl.BlockSpec((_EB, HID), lambda i: (i, 0)),
            pl.BlockSpec((_EB, HID), lambda i: (i, 0)),
        ],
        out_shape=[
            jax.ShapeDtypeStruct((ea.shape[0], HID), jnp.float32),
            jax.ShapeDtypeStruct((ea.shape[0], HID), jnp.float32),
        ],
    )(ea, we, be.reshape(1, HID), g, w, b.reshape(1, HID),
      lg.reshape(1, HID), lb.reshape(1, HID))


def _edge_update_last_body(e_ref, g_ref, w_ref, b_ref, en_ref):
    en = (jnp.dot(e_ref[...], w_ref[...], preferred_element_type=jnp.float32)
          + g_ref[...] + b_ref[...])
    en_ref[...] = en


def _edge_update_last(e, g, w, b):
    return pl.pallas_call(
        _edge_update_last_body,
        grid=(e.shape[0] // _EB,),
        in_specs=[
            pl.BlockSpec((_EB, HID), lambda i: (i, 0)),
            pl.BlockSpec((_EB, HID), lambda i: (i, 0)),
            pl.BlockSpec((HID, HID), lambda i: (0, 0)),
            pl.BlockSpec((1, HID), lambda i: (0, 0)),
        ],
        out_specs=pl.BlockSpec((_EB, HID), lambda i: (i, 0)),
        out_shape=jax.ShapeDtypeStruct((e.shape[0], HID), jnp.float32),
    )(e, g, w, b.reshape(1, HID))


def _node_update_body(x_ref, agg_ref, agg2_ref, wm1_ref, wm2_ref, bm_ref,
                      wx1_ref, wx2_ref, bx_ref, lg_ref, lb_ref,
                      ws_ref, wr_ref, xo_ref, ps_ref, pr_ref):
    x = x_ref[...]
    agg_r = agg_ref[0] + agg2_ref[0]
    agg_s = agg_ref[1] + agg2_ref[1]
    msg = (jnp.dot(agg_r, wm1_ref[...], preferred_element_type=jnp.float32)
           + jnp.dot(agg_s, wm2_ref[...], preferred_element_type=jnp.float32)
           + bm_ref[...])
    xn = (jnp.dot(x, wx1_ref[...], preferred_element_type=jnp.float32)
          + jnp.dot(msg, wx2_ref[...], preferred_element_type=jnp.float32)
          + bx_ref[...])
    xo = _ln(x + xn, lg_ref[...], lb_ref[...])
    xo_ref[...] = xo
    ps_ref[...] = jnp.dot(xo, ws_ref[...], preferred_element_type=jnp.float32)
    pr_ref[...] = jnp.dot(xo, wr_ref[...], preferred_element_type=jnp.float32)


def _node_update(x, agg, agg2, wm1, wm2, bm, wx1, wx2, bx, lg, lb, ws, wr):
    return pl.pallas_call(
        _node_update_body,
        out_shape=[
            jax.ShapeDtypeStruct((N_NODES, HID), jnp.float32),
            jax.ShapeDtypeStruct((N_NODES, HID), jnp.float32),
            jax.ShapeDtypeStruct((N_NODES, HID), jnp.float32),
        ],
    )(x, agg, agg2, wm1, wm2, bm.reshape(1, HID), wx1, wx2, bx.reshape(1, HID),
      lg.reshape(1, HID), lb.reshape(1, HID), ws, wr)


def _head_body(x_ref, w0_ref, b0_ref, w1_ref, b1_ref, w2_ref, b2_ref, o_ref):
    h = jnp.dot(x_ref[...], w0_ref[...], preferred_element_type=jnp.float32)
    h = jax.nn.silu(h + b0_ref[...])
    h = jnp.dot(h, w1_ref[...], preferred_element_type=jnp.float32)
    h = jax.nn.silu(h + b1_ref[...])
    raw = jnp.dot(h, w2_ref[...], preferred_element_type=jnp.float32) + b2_ref[...]
    kappa = jax.nn.sigmoid(raw[:, 0:1]) * 2.0
    a = jnp.exp(raw[:, 1:6]) * 100.0
    bb = 35.0 * jnp.exp(jnp.log(kappa) * (-1.0 / 3.0))
    o_ref[...] = jnp.concatenate([kappa, a, bb], axis=-1)


def _head(x, f0, f1, f2):
    return pl.pallas_call(
        _head_body,
        out_shape=jax.ShapeDtypeStruct((N_NODES, 7), jnp.float32),
    )(x, f0["W"], f0["b"].reshape(1, 64), f1["W"], f1["b"].reshape(1, 32),
      f2["W"], f2["b"].reshape(1, 6))


_EH = N_EDGES // 2
_GCH = 40
_SSCH = 40
_gather_h = _make_gather(_EH, _GCH)
_segsum_h = _make_segsum(_EH, _SSCH)


def kernel(node_features, edge_attr, senders, receivers, params):
    p = params
    layers = p["layers"]

    s_h = (senders[:_EH], senders[_EH:])
    r_h = (receivers[:_EH], receivers[_EH:])
    ea_h = (edge_attr[:_EH], edge_attr[_EH:])
    s3d = tuple(a.reshape(NW, _EH // NW // _GCH, _GCH) for a in s_h)
    r3d = tuple(a.reshape(NW, _EH // NW // _GCH, _GCH) for a in r_h)
    idxcat = tuple(jnp.concatenate([r_h[h], s_h[h]]) for h in range(2))

    w0 = layers[0]["edge"]["W"]
    x, ps, pr = _node_embed(node_features, p["node_embed"]["W"],
                            p["node_embed"]["b"],
                            w0[HID:2 * HID], w0[2 * HID:])

    e = [None, None]
    zeros_w = jnp.zeros((HID, HID), jnp.float32)
    for li, lp in enumerate(layers):
        we = lp["edge"]["W"]
        g = [None, None]
        e_new = [None, None]
        agg = [None, None]
        for h in range(2):
            g[h] = _gather_h(ps, pr, s3d[h], r3d[h])
        for h in range(2):
            if li == 0:
                e[h], e_new[h] = _edge_update_first(
                    ea_h[h], p["edge_embed"]["W"], p["edge_embed"]["b"],
                    g[h], we[:HID], lp["edge"]["b"],
                    lp["ln_e"]["g"], lp["ln_e"]["b"])
            elif li + 1 == len(layers):
                e_new[h] = _edge_update_last(e[h], g[h], we[:HID],
                                             lp["edge"]["b"])
            else:
                e[h], e_new[h] = _edge_update(e[h], g[h], we[:HID],
                                              lp["edge"]["b"],
                                              lp["ln_e"]["g"], lp["ln_e"]["b"])
            agg[h] = _segsum_h(e_new[h], idxcat[h])
        wm = lp["msg"]["W"]
        wn = lp["node"]["W"]
        if li + 1 < len(layers):
            wnext = layers[li + 1]["edge"]["W"]
            ws_n, wr_n = wnext[HID:2 * HID], wnext[2 * HID:]
        else:
            ws_n, wr_n = zeros_w, zeros_w
        x, ps, pr = _node_update(x, agg[0], agg[1],
                                 wm[:HID], wm[HID:], lp["msg"]["b"],
                                 wn[:HID], wn[HID:], lp["node"]["b"],
                                 lp["ln_n"]["g"], lp["ln_n"]["b"], ws_n, wr_n)

    return _head(x, p["ff"][0], p["ff"][1], p["ff"][2])


# speedup vs baseline: 1.1644x; 1.1644x over previous
import functools

import jax
import jax.numpy as jnp
from jax import lax
from jax.experimental import pallas as pl
from jax.experimental.pallas import tpu as pltpu
from jax.experimental.pallas import tpu_sc as plsc

N_NODES = 10000
N_EDGES = 320000
HID = 128
LANES = 16
NC = 2
NS = 16
NW = NC * NS


_sc_mesh = plsc.VectorSubcoreMesh(core_axis_name="c", subcore_axis_name="s",
                                  num_cores=NC, num_subcores=NS)


def _make_gather(n_edges, ch):
    epw = n_edges // NW
    nch = epw // ch
    assert epw % ch == 0 and ch % 8 == 0 and ch <= 128

    def body(ps_hbm, pr_hbm, s2d_hbm, r2d_hbm, out_hbm,
             idx_s, idx_r, rows_s0, rows_r0, rows_s1, rows_r1,
             obuf0, obuf1,
             sem_s0, sem_r0, sem_s1, sem_r1, sem_w0, sem_w1):
        wid = lax.axis_index("s") * NC + lax.axis_index("c")
        base = wid * epw

        rows_s = (rows_s0, rows_s1)
        rows_r = (rows_r0, rows_r1)
        obuf = (obuf0, obuf1)
        sem_s = (sem_s0, sem_s1)
        sem_r = (sem_r0, sem_r1)
        sem_w = (sem_w0, sem_w1)

        pltpu.sync_copy(s2d_hbm.at[wid], idx_s)
        pltpu.sync_copy(r2d_hbm.at[wid], idx_r)

        def g_start(j, b):
            pltpu.make_async_copy(ps_hbm.at[idx_s.at[j]], rows_s[b], sem_s[b]).start()
            pltpu.make_async_copy(pr_hbm.at[idx_r.at[j]], rows_r[b], sem_r[b]).start()

        def g_wait(j, b):
            pltpu.make_async_copy(ps_hbm.at[idx_s.at[j]], rows_s[b], sem_s[b]).wait()
            pltpu.make_async_copy(pr_hbm.at[idx_r.at[j]], rows_r[b], sem_r[b]).wait()

        def w_start(j, b):
            pltpu.make_async_copy(obuf[b], out_hbm.at[pl.ds(base + j * ch, ch)],
                                  sem_w[b]).start()

        def w_wait(j, b):
            pltpu.make_async_copy(obuf[b], out_hbm.at[pl.ds(base + j * ch, ch)],
                                  sem_w[b]).wait()

        g_start(0, 0)

        def pair(jj, carry):
            for b in range(2):
                j = 2 * jj + b

                @pl.when(j + 1 < nch)
                def _():
                    g_start(j + 1, 1 - b)

                @pl.when(j < nch)
                def _():
                    g_wait(j, b)

                    @pl.when(j >= 2)
                    def _():
                        w_wait(j - 2, b)

                    def addrow(i, c2):
                        for k in range(HID // LANES):
                            sl = pl.ds(k * LANES, LANES)
                            obuf[b][i, sl] = rows_s[b][i, sl] + rows_r[b][i, sl]
                        return c2

                    lax.fori_loop(0, ch, addrow, 0)
                    w_start(j, b)
            return carry

        lax.fori_loop(0, (nch + 1) // 2, pair, 0)
        w_wait(nch - 2, nch % 2)
        w_wait(nch - 1, 1 - nch % 2)

    return functools.partial(
        pl.kernel,
        out_type=jax.ShapeDtypeStruct((n_edges, HID), jnp.float32),
        mesh=_sc_mesh,
        scratch_types=[
            pltpu.VMEM((nch, ch), jnp.int32),
            pltpu.VMEM((nch, ch), jnp.int32),
            pltpu.VMEM((ch, HID), jnp.float32),
            pltpu.VMEM((ch, HID), jnp.float32),
            pltpu.VMEM((ch, HID), jnp.float32),
            pltpu.VMEM((ch, HID), jnp.float32),
            pltpu.VMEM((ch, HID), jnp.float32),
            pltpu.VMEM((ch, HID), jnp.float32),
            pltpu.SemaphoreType.DMA,
            pltpu.SemaphoreType.DMA,
            pltpu.SemaphoreType.DMA,
            pltpu.SemaphoreType.DMA,
            pltpu.SemaphoreType.DMA,
            pltpu.SemaphoreType.DMA,
        ],
    )(body)


_NPAD = 10240
_RPT = _NPAD // NS


def _make_segsum(n_edges, ch):
    ept = n_edges // NS
    snch = ept // ch
    assert ept % ch == 0 and snch % 2 == 0 and ch % 8 == 0 and ch <= 128

    def body(enew_hbm, idxcat_hbm, out_hbm, idx0, idx1, rows0, rows1, acc,
             sem_i0, sem_i1, sem_l0, sem_l1, sem_a0, sem_a1):
        c = lax.axis_index("c")
        s = lax.axis_index("s")

        idx = (idx0, idx1)
        rows = (rows0, rows1)
        sem_i = (sem_i0, sem_i1)
        sem_l = (sem_l0, sem_l1)
        sem_a = (sem_a0, sem_a1)

        zv = jnp.zeros((LANES,), jnp.float32)

        def zrow(i, c2):
            for k in range(HID // LANES):
                rows0[i, pl.ds(k * LANES, LANES)] = zv
            return c2

        lax.fori_loop(0, ch, zrow, 0)
        for k in range(_RPT // ch):
            pltpu.sync_copy(rows0, acc.at[pl.ds(s * _RPT + k * ch, ch)])
        plsc.subcore_barrier()

        def l_start(j, b):
            pltpu.make_async_copy(
                idxcat_hbm.at[pl.ds(c * n_edges + s * ept + j * ch, ch)],
                idx[b], sem_i[b]).start()
            pltpu.make_async_copy(enew_hbm.at[pl.ds(s * ept + j * ch, ch)],
                                  rows[b], sem_l[b]).start()

        def l_wait(j, b):
            pltpu.make_async_copy(
                idxcat_hbm.at[pl.ds(c * n_edges + s * ept + j * ch, ch)],
                idx[b], sem_i[b]).wait()
            pltpu.make_async_copy(enew_hbm.at[pl.ds(s * ept + j * ch, ch)],
                                  rows[b], sem_l[b]).wait()

        def a_start(j, b):
            pltpu.async_copy(rows[b], acc.at[idx[b]], sem_a[b], add=True)

        def a_wait(j, b):
            pltpu.make_async_copy(rows[b], acc.at[idx[b]], sem_a[b]).wait()

        l_start(0, 0)

        def pair(jj, carry):
            for b in range(2):
                j = 2 * jj + b

                @pl.when(j + 1 < snch)
                def _():
                    @pl.when(j >= 1)
                    def _():
                        a_wait(j - 1, 1 - b)

                    l_start(j + 1, 1 - b)

                l_wait(j, b)
                a_start(j, b)
            return carry

        lax.fori_loop(0, snch // 2, pair, 0)
        a_wait(snch - 2, 0)
        a_wait(snch - 1, 1)
        plsc.subcore_barrier()

        @pl.when(s < NS - 1)
        def _():
            pltpu.sync_copy(acc.at[pl.ds(s * _RPT, _RPT)],
                            out_hbm.at[c, pl.ds(s * _RPT, _RPT)])

        @pl.when(s == NS - 1)
        def _():
            last = N_NODES - (NS - 1) * _RPT
            pltpu.sync_copy(acc.at[pl.ds((NS - 1) * _RPT, last)],
                            out_hbm.at[c, pl.ds((NS - 1) * _RPT, last)])

    return functools.partial(
        pl.kernel,
        out_type=jax.ShapeDtypeStruct((2, N_NODES, HID), jnp.float32),
        mesh=_sc_mesh,
        scratch_types=[
            pltpu.VMEM((ch,), jnp.int32),
            pltpu.VMEM((ch,), jnp.int32),
            pltpu.VMEM((ch, HID), jnp.float32),
            pltpu.VMEM((ch, HID), jnp.float32),
            pltpu.VMEM_SHARED((_NPAD, HID), jnp.float32),
            pltpu.SemaphoreType.DMA,
            pltpu.SemaphoreType.DMA,
            pltpu.SemaphoreType.DMA,
            pltpu.SemaphoreType.DMA,
            pltpu.SemaphoreType.DMA,
            pltpu.SemaphoreType.DMA,
        ],
    )(body)


def _node_embed_body(nf_ref, w_ref, b_ref, ws_ref, wr_ref, x_ref, ps_ref, pr_ref):
    x = jnp.dot(nf_ref[...], w_ref[...], preferred_element_type=jnp.float32)
    x = x + b_ref[...]
    x_ref[...] = x
    ps_ref[...] = jnp.dot(x, ws_ref[...], preferred_element_type=jnp.float32)
    pr_ref[...] = jnp.dot(x, wr_ref[...], preferred_element_type=jnp.float32)


def _node_embed(nf, w, b, ws, wr):
    return pl.pallas_call(
        _node_embed_body,
        out_shape=[
            jax.ShapeDtypeStruct((N_NODES, HID), jnp.float32),
            jax.ShapeDtypeStruct((N_NODES, HID), jnp.float32),
            jax.ShapeDtypeStruct((N_NODES, HID), jnp.float32),
        ],
    )(nf, w, b.reshape(1, HID), ws, wr)


_EB = 2560


def _ln(r, g, b):
    m = jnp.mean(r, axis=-1, keepdims=True)
    d = r - m
    v = jnp.mean(d * d, axis=-1, keepdims=True)
    return d * lax.rsqrt(v + 1e-6) * g + b


def _edge_update_body(e_ref, g_ref, w_ref, b_ref, lg_ref, lb_ref,
                      eo_ref, en_ref):
    e = e_ref[...]
    en = (jnp.dot(e, w_ref[...], preferred_element_type=jnp.float32)
          + g_ref[...] + b_ref[...])
    en_ref[...] = en
    eo_ref[...] = _ln(e + en, lg_ref[...], lb_ref[...])


def _edge_update(e, g, w, b, lg, lb):
    return pl.pallas_call(
        _edge_update_body,
        grid=(e.shape[0] // _EB,),
        in_specs=[
            pl.BlockSpec((_EB, HID), lambda i: (i, 0)),
            pl.BlockSpec((_EB, HID), lambda i: (i, 0)),
            pl.BlockSpec((HID, HID), lambda i: (0, 0)),
            pl.BlockSpec((1, HID), lambda i: (0, 0)),
            pl.BlockSpec((1, HID), lambda i: (0, 0)),
            pl.BlockSpec((1, HID), lambda i: (0, 0)),
        ],
        out_specs=[
            pl.BlockSpec((_EB, HID), lambda i: (i, 0)),
            pl.BlockSpec((_EB, HID), lambda i: (i, 0)),
        ],
        out_shape=[
            jax.ShapeDtypeStruct((e.shape[0], HID), jnp.float32),
            jax.ShapeDtypeStruct((e.shape[0], HID), jnp.float32),
        ],
    )(e, g, w, b.reshape(1, HID), lg.reshape(1, HID), lb.reshape(1, HID))


def _edge_update_first_body(ea_ref, we_ref, be_ref, g_ref, w_ref, b_ref,
                            lg_ref, lb_ref, eo_ref, en_ref):
    e = (jnp.dot(ea_ref[...], we_ref[...], preferred_element_type=jnp.float32)
         + be_ref[...])
    en = (jnp.dot(e, w_ref[...], preferred_element_type=jnp.float32)
          + g_ref[...] + b_ref[...])
    en_ref[...] = en
    eo_ref[...] = _ln(e + en, lg_ref[...], lb_ref[...])


def _edge_update_first(ea, we, be, g, w, b, lg, lb):
    return pl.pallas_call(
        _edge_update_first_body,
        grid=(ea.shape[0] // _EB,),
        in_specs=[
            pl.BlockSpec((_EB, 16), lambda i: (i, 0)),
            pl.BlockSpec((16, HID), lambda i: (0, 0)),
            pl.BlockSpec((1, HID), lambda i: (0, 0)),
            pl.BlockSpec((_EB, HID), lambda i: (i, 0)),
            pl.BlockSpec((HID, HID), lambda i: (0, 0)),
            pl.BlockSpec((1, HID), lambda i: (0, 0)),
            pl.BlockSpec((1, HID), lambda i: (0, 0)),
            pl.BlockSpec((1, HID), lambda i: (0, 0)),
        ],
        out_specs=[
            pl.BlockSpec((_EB, HID), lambda i: (i, 0)),
            pl.BlockSpec((_EB, HID), lambda i: (i, 0)),
        ],
        out_shape=[
            jax.ShapeDtypeStruct((ea.shape[0], HID), jnp.float32),
            jax.ShapeDtypeStruct((ea.shape[0], HID), jnp.float32),
        ],
    )(ea, we, be.reshape(1, HID), g, w, b.reshape(1, HID),
      lg.reshape(1, HID), lb.reshape(1, HID))


def _edge_update_last_body(e_ref, g_ref, w_ref, b_ref, en_ref):
    en = (jnp.dot(e_ref[...], w_ref[...], preferred_element_type=jnp.float32)
          + g_ref[...] + b_ref[...])
    en_ref[...] = en


def _edge_update_last(e, g, w, b):
    return pl.pallas_call(
        _edge_update_last_body,
        grid=(e.shape[0] // _EB,),
        in_specs=[
            pl.BlockSpec((_EB, HID), lambda i: (i, 0)),
            pl.BlockSpec((_EB, HID), lambda i: (i, 0)),
            pl.BlockSpec((HID, HID), lambda i: (0, 0)),
            pl.BlockSpec((1, HID), lambda i: (0, 0)),
        ],
        out_specs=pl.BlockSpec((_EB, HID), lambda i: (i, 0)),
        out_shape=jax.ShapeDtypeStruct((e.shape[0], HID), jnp.float32),
    )(e, g, w, b.reshape(1, HID))


def _node_update_body(x_ref, agg_ref, agg2_ref, wm1_ref, wm2_ref, bm_ref,
                      wx1_ref, wx2_ref, bx_ref, lg_ref, lb_ref,
                      ws_ref, wr_ref, xo_ref, ps_ref, pr_ref):
    x = x_ref[...]
    agg_r = agg_ref[0] + agg2_ref[0]
    agg_s = agg_ref[1] + agg2_ref[1]
    msg = (jnp.dot(agg_r, wm1_ref[...], preferred_element_type=jnp.float32)
           + jnp.dot(agg_s, wm2_ref[...], preferred_element_type=jnp.float32)
           + bm_ref[...])
    xn = (jnp.dot(x, wx1_ref[...], preferred_element_type=jnp.float32)
          + jnp.dot(msg, wx2_ref[...], preferred_element_type=jnp.float32)
          + bx_ref[...])
    xo = _ln(x + xn, lg_ref[...], lb_ref[...])
    xo_ref[...] = xo
    ps_ref[...] = jnp.dot(xo, ws_ref[...], preferred_element_type=jnp.float32)
    pr_ref[...] = jnp.dot(xo, wr_ref[...], preferred_element_type=jnp.float32)


def _node_update(x, agg, agg2, wm1, wm2, bm, wx1, wx2, bx, lg, lb, ws, wr):
    return pl.pallas_call(
        _node_update_body,
        out_shape=[
            jax.ShapeDtypeStruct((N_NODES, HID), jnp.float32),
            jax.ShapeDtypeStruct((N_NODES, HID), jnp.float32),
            jax.ShapeDtypeStruct((N_NODES, HID), jnp.float32),
        ],
    )(x, agg, agg2, wm1, wm2, bm.reshape(1, HID), wx1, wx2, bx.reshape(1, HID),
      lg.reshape(1, HID), lb.reshape(1, HID), ws, wr)


def _head_body(x_ref, w0_ref, b0_ref, w1_ref, b1_ref, w2_ref, b2_ref, o_ref):
    h = jnp.dot(x_ref[...], w0_ref[...], preferred_element_type=jnp.float32)
    h = jax.nn.silu(h + b0_ref[...])
    h = jnp.dot(h, w1_ref[...], preferred_element_type=jnp.float32)
    h = jax.nn.silu(h + b1_ref[...])
    raw = jnp.dot(h, w2_ref[...], preferred_element_type=jnp.float32) + b2_ref[...]
    kappa = jax.nn.sigmoid(raw[:, 0:1]) * 2.0
    a = jnp.exp(raw[:, 1:6]) * 100.0
    bb = 35.0 * jnp.exp(jnp.log(kappa) * (-1.0 / 3.0))
    o_ref[...] = jnp.concatenate([kappa, a, bb], axis=-1)


def _head(x, f0, f1, f2):
    return pl.pallas_call(
        _head_body,
        out_shape=jax.ShapeDtypeStruct((N_NODES, 7), jnp.float32),
    )(x, f0["W"], f0["b"].reshape(1, 64), f1["W"], f1["b"].reshape(1, 32),
      f2["W"], f2["b"].reshape(1, 6))


_EHS = (161280, 158720)
_GCH = 80
_SSCH = 80
_gather_h = tuple(_make_gather(n, _GCH) for n in _EHS)
_segsum_h = tuple(_make_segsum(n, _SSCH) for n in _EHS)


def kernel(node_features, edge_attr, senders, receivers, params):
    p = params
    layers = p["layers"]

    cut = _EHS[0]
    s_h = (senders[:cut], senders[cut:])
    r_h = (receivers[:cut], receivers[cut:])
    ea_h = (edge_attr[:cut], edge_attr[cut:])
    s3d = tuple(a.reshape(NW, _EHS[h] // NW // _GCH, _GCH)
                for h, a in enumerate(s_h))
    r3d = tuple(a.reshape(NW, _EHS[h] // NW // _GCH, _GCH)
                for h, a in enumerate(r_h))
    idxcat = tuple(jnp.concatenate([r_h[h], s_h[h]]) for h in range(2))

    w0 = layers[0]["edge"]["W"]
    x, ps, pr = _node_embed(node_features, p["node_embed"]["W"],
                            p["node_embed"]["b"],
                            w0[HID:2 * HID], w0[2 * HID:])

    e = [None, None]
    zeros_w = jnp.zeros((HID, HID), jnp.float32)
    for li, lp in enumerate(layers):
        we = lp["edge"]["W"]
        g = [None, None]
        e_new = [None, None]
        agg = [None, None]
        for h in range(2):
            g[h] = _gather_h[h](ps, pr, s3d[h], r3d[h])
        for h in range(2):
            if li == 0:
                e[h], e_new[h] = _edge_update_first(
                    ea_h[h], p["edge_embed"]["W"], p["edge_embed"]["b"],
                    g[h], we[:HID], lp["edge"]["b"],
                    lp["ln_e"]["g"], lp["ln_e"]["b"])
            elif li + 1 == len(layers):
                e_new[h] = _edge_update_last(e[h], g[h], we[:HID],
                                             lp["edge"]["b"])
            else:
                e[h], e_new[h] = _edge_update(e[h], g[h], we[:HID],
                                              lp["edge"]["b"],
                                              lp["ln_e"]["g"], lp["ln_e"]["b"])
            agg[h] = _segsum_h[h](e_new[h], idxcat[h])
        wm = lp["msg"]["W"]
        wn = lp["node"]["W"]
        if li + 1 < len(layers):
            wnext = layers[li + 1]["edge"]["W"]
            ws_n, wr_n = wnext[HID:2 * HID], wnext[2 * HID:]
        else:
            ws_n, wr_n = zeros_w, zeros_w
        x, ps, pr = _node_update(x, agg[0], agg[1],
                                 wm[:HID], wm[HID:], lp["msg"]["b"],
                                 wn[:HID], wn[HID:], lp["node"]["b"],
                                 lp["ln_n"]["g"], lp["ln_n"]["b"], ws_n, wr_n)

    return _head(x, p["ff"][0], p["ff"][1], p["ff"][2])
